# baseline (speedup 1.0000x reference)
import functools

import jax
import jax.numpy as jnp
from jax import lax
from jax.experimental import pallas as pl
from jax.experimental.pallas import tpu as pltpu


def _router_body(E, RB, h_ref, wr_ref, logits_ref, e2_ref, w2_ref, rank_ref,
                 cnt_ref, aux_ref, runcnt, impacc):
    i = pl.program_id(0)
    nsteps = pl.num_programs(0)

    @pl.when(i == 0)
    def _():
        runcnt[...] = jnp.zeros_like(runcnt)
        impacc[...] = jnp.zeros_like(impacc)

    h = h_ref[...].astype(jnp.bfloat16)
    logits = lax.dot_general(h, wr_ref[...].astype(jnp.bfloat16),
                             (((1,), (0,)), ((), ())),
                             preferred_element_type=jnp.float32)
    logits_ref[...] = logits

    m = jnp.max(logits, axis=1, keepdims=True)
    ex = jnp.exp(logits - m)
    probs = ex / jnp.sum(ex, axis=1, keepdims=True)

    iota_e = lax.broadcasted_iota(jnp.int32, probs.shape, 1)
    m1 = jnp.max(probs, axis=1, keepdims=True)
    a1 = jnp.min(jnp.where(probs == m1, iota_e, E), axis=1, keepdims=True)
    masked = jnp.where(iota_e == a1, -1.0, probs)
    m2 = jnp.max(masked, axis=1, keepdims=True)
    a2 = jnp.min(jnp.where(masked == m2, iota_e, E), axis=1, keepdims=True)

    e2 = jnp.concatenate([a1, a2], axis=1)
    w2 = jnp.concatenate([m1, m2], axis=1)
    e2_ref[...] = e2
    w2_ref[...] = w2

    iota_e3 = lax.broadcasted_iota(jnp.int32, (1, 1, E), 2)
    oh = (e2[:, :, None] == iota_e3).astype(jnp.float32)
    oh_f = oh.reshape(RB * 2, E)
    n = RB * 2
    tri = (lax.broadcasted_iota(jnp.int32, (n, n), 0)
           >= lax.broadcasted_iota(jnp.int32, (n, n), 1)).astype(jnp.bfloat16)
    c = lax.dot_general(tri, oh_f.astype(jnp.bfloat16),
                        (((1,), (0,)), ((), ())),
                        preferred_element_type=jnp.float32)
    c = c.astype(jnp.int32)
    oh_i = oh_f.astype(jnp.int32)
    rank_f = jnp.sum(oh_i * (runcnt[...] + c - 1), axis=1)
    rank_ref[...] = rank_f.reshape(RB, 2)

    runcnt[...] = runcnt[...] + c[-1:, :]
    impacc[...] = impacc[...] + jnp.sum(probs, axis=0, keepdims=True)

    @pl.when(i == nsteps - 1)
    def _():
        cnt = runcnt[...]
        cnt_ref[...] = cnt
        eps = 1e-09
        imp = impacc[...]
        impn = imp / (jnp.sum(imp) + eps)
        cntf = cnt.astype(jnp.float32)
        ldn = cntf / (jnp.sum(cntf) + eps)
        aux_ref[...] = jnp.full((1, 1), E, jnp.float32) * jnp.sum(impn * ldn)


def _router(h, W_router, RB):
    T, D = h.shape
    E = W_router.shape[1]
    grid = (T // RB,)
    return pl.pallas_call(
        functools.partial(_router_body, E, RB),
        grid=grid,
        in_specs=[
            pl.BlockSpec((RB, D), lambda i: (i, 0)),
            pl.BlockSpec((D, E), lambda i: (0, 0)),
        ],
        out_specs=[
            pl.BlockSpec((RB, E), lambda i: (i, 0)),
            pl.BlockSpec((RB, 2), lambda i: (i, 0)),
            pl.BlockSpec((RB, 2), lambda i: (i, 0)),
            pl.BlockSpec((RB, 2), lambda i: (i, 0)),
            pl.BlockSpec((1, E), lambda i: (0, 0)),
            pl.BlockSpec((1, 1), lambda i: (0, 0)),
        ],
        out_shape=[
            jax.ShapeDtypeStruct((T, E), jnp.float32),
            jax.ShapeDtypeStruct((T, 2), jnp.int32),
            jax.ShapeDtypeStruct((T, 2), jnp.float32),
            jax.ShapeDtypeStruct((T, 2), jnp.int32),
            jax.ShapeDtypeStruct((1, E), jnp.int32),
            jax.ShapeDtypeStruct((1, 1), jnp.float32),
        ],
        scratch_shapes=[
            pltpu.VMEM((1, E), jnp.int32),
            pltpu.VMEM((1, E), jnp.float32),
        ],
        compiler_params=pltpu.CompilerParams(
            dimension_semantics=("arbitrary",)),
    )(h, W_router)


def _meta_body(E, BT, NB, cnt_ref, e2_ref, rank_ref, pflat_ref, blk_ref):
    cnt = cnt_ref[...]
    padded = ((cnt + BT - 1) // BT) * BT
    ii = lax.broadcasted_iota(jnp.int32, (1, E, E), 1)
    jj = lax.broadcasted_iota(jnp.int32, (1, E, E), 2)
    csum = jnp.sum(jnp.where(jj <= ii, padded[:, None, :], 0), axis=2)
    start = csum - padded

    e2 = e2_ref[...]
    iota_e = lax.broadcasted_iota(jnp.int32, (1, 1, E), 2)
    oh = (e2[:, :, None] == iota_e).astype(jnp.int32)
    st = jnp.sum(oh * start[:, None, :], axis=2)
    pflat_ref[...] = st + rank_ref[...]

    bidx = lax.broadcasted_iota(jnp.int32, (1, NB), 1) * BT
    ge = (bidx[:, :, None] >= csum[:, None, :]).astype(jnp.int32)
    blk_ref[...] = jnp.minimum(jnp.sum(ge, axis=2), E - 1)


def _meta(cnt, e2, rank, BT, NB):
    T = e2.shape[0]
    E = cnt.shape[1]
    return pl.pallas_call(
        functools.partial(_meta_body, E, BT, NB),
        out_shape=[
            jax.ShapeDtypeStruct((T, 2), jnp.int32),
            jax.ShapeDtypeStruct((1, NB), jnp.int32),
        ],
    )(cnt, e2, rank)


def _shared_body(E, NJ, hb_ref, wsg_ref, wsd_ref, e2_ref, wpgt_ref, y_ref,
                 acc):
    j = pl.program_id(1)

    @pl.when(j == 0)
    def _():
        acc[...] = jnp.zeros_like(acc)

    a = lax.dot_general(hb_ref[...], wsg_ref[...], (((1,), (0,)), ((), ())),
                        preferred_element_type=jnp.float32)
    a = jnp.maximum(a, 0.0).astype(jnp.bfloat16)
    acc[...] = acc[...] + lax.dot_general(
        a, wsd_ref[...], (((1,), (0,)), ((), ())),
        preferred_element_type=jnp.float32)

    @pl.when(j == NJ - 1)
    def _():
        hs = acc[...]
        p = lax.dot_general(hs.astype(jnp.bfloat16), wpgt_ref[...],
                            (((1,), (0,)), ((), ())),
                            preferred_element_type=jnp.float32)
        p = jax.nn.sigmoid(p)
        e2 = e2_ref[...]
        iota_e2 = lax.broadcasted_iota(jnp.int32, p.shape, 1)
        for k in range(2):
            psel = jnp.sum(
                jnp.where(e2[:, k:k + 1] == iota_e2, p, 0.0),
                axis=1, keepdims=True)
            y_ref[:, k, :] = (psel * hs).astype(jnp.bfloat16)


def _shared(hb, Wsg, Wsd, e2, Wpgt, TB, DSB):
    T, D = hb.shape
    DS = Wsg.shape[1]
    E = Wpgt.shape[1]
    NJ = DS // DSB
    grid = (T // TB, NJ)
    return pl.pallas_call(
        functools.partial(_shared_body, E, NJ),
        grid=grid,
        in_specs=[
            pl.BlockSpec((TB, D), lambda i, j: (i, 0)),
            pl.BlockSpec((D, DSB), lambda i, j: (0, j)),
            pl.BlockSpec((DSB, D), lambda i, j: (j, 0)),
            pl.BlockSpec((TB, 2), lambda i, j: (i, 0)),
            pl.BlockSpec((D, E), lambda i, j: (0, 0)),
        ],
        out_specs=pl.BlockSpec((TB, 2, D), lambda i, j: (i, 0, 0)),
        out_shape=jax.ShapeDtypeStruct((T, 2, D), jnp.bfloat16),
        scratch_shapes=[pltpu.VMEM((TB, D), jnp.float32)],
        compiler_params=pltpu.CompilerParams(
            dimension_semantics=("parallel", "arbitrary")),
    )(hb, Wsg, Wsd, e2, Wpgt)


def _scatter_body(p_ref, y_ref, o_ref):
    o_ref[...] = y_ref[...]


def _dispatch_scatter(y_flat, pflat, NMAX):
    TK, D = y_flat.shape
    grid_spec = pltpu.PrefetchScalarGridSpec(
        num_scalar_prefetch=1,
        grid=(TK,),
        in_specs=[pl.BlockSpec((1, 1, D), lambda a, p: (a, 0, 0))],
        out_specs=pl.BlockSpec((1, 1, D), lambda a, p: (p[a], 0, 0)),
    )
    out = pl.pallas_call(
        _scatter_body,
        grid_spec=grid_spec,
        out_shape=jax.ShapeDtypeStruct((NMAX, 1, D), jnp.bfloat16),
        compiler_params=pltpu.CompilerParams(
            dimension_semantics=("arbitrary",)),
    )(pflat, y_flat.reshape(TK, 1, D))
    return out.reshape(NMAX, D)


def _expert_body(be_ref, x_ref, wg_ref, wd_ref, o_ref):
    x = x_ref[...]
    wg = wg_ref[0].astype(jnp.bfloat16)
    t = lax.dot_general(x, wg, (((1,), (0,)), ((), ())),
                        preferred_element_type=jnp.float32)
    t = jnp.maximum(t, 0.0).astype(jnp.bfloat16)
    o = lax.dot_general(t, wd_ref[0].astype(jnp.bfloat16),
                        (((1,), (0,)), ((), ())),
                        preferred_element_type=jnp.float32)
    o_ref[...] = o.astype(jnp.bfloat16)


def _expert_ffn(blk_e, x_d, W_gate, W_down, BT):
    NMAX, D = x_d.shape
    E, _, DP = W_gate.shape
    NB = NMAX // BT
    grid_spec = pltpu.PrefetchScalarGridSpec(
        num_scalar_prefetch=1,
        grid=(NB,),
        in_specs=[
            pl.BlockSpec((BT, D), lambda i, be: (i, 0)),
            pl.BlockSpec((1, D, DP), lambda i, be: (be[i], 0, 0)),
            pl.BlockSpec((1, DP, D), lambda i, be: (be[i], 0, 0)),
        ],
        out_specs=pl.BlockSpec((BT, D), lambda i, be: (i, 0)),
    )
    return pl.pallas_call(
        _expert_body,
        grid_spec=grid_spec,
        out_shape=jax.ShapeDtypeStruct((NMAX, D), jnp.bfloat16),
        compiler_params=pltpu.CompilerParams(
            dimension_semantics=("arbitrary",)),
    )(blk_e, x_d, W_gate, W_down)


def _combine_body(p_ref, a_ref, b_ref, w_ref, o_ref):
    w = w_ref[...]
    o_ref[...] = (w[0, 0, 0] * a_ref[...].astype(jnp.float32)
                  + w[0, 0, 1] * b_ref[...].astype(jnp.float32))


def _combine(out_d, pflat, w2, T):
    NMAX, D = out_d.shape
    w3 = w2.reshape(T, 1, 2)
    od3 = out_d.reshape(NMAX, 1, D)
    grid_spec = pltpu.PrefetchScalarGridSpec(
        num_scalar_prefetch=1,
        grid=(T,),
        in_specs=[
            pl.BlockSpec((1, 1, D), lambda t, p: (p[2 * t], 0, 0)),
            pl.BlockSpec((1, 1, D), lambda t, p: (p[2 * t + 1], 0, 0)),
            pl.BlockSpec((1, 1, 2), lambda t, p: (t, 0, 0)),
        ],
        out_specs=pl.BlockSpec((1, 1, D), lambda t, p: (t, 0, 0)),
    )
    out = pl.pallas_call(
        _combine_body,
        grid_spec=grid_spec,
        out_shape=jax.ShapeDtypeStruct((T, 1, D), jnp.float32),
        compiler_params=pltpu.CompilerParams(
            dimension_semantics=("arbitrary",)),
    )(pflat, od3, od3, w3)
    return out.reshape(T, D)


def kernel(hidden_states, W_router, W_shared_gate, W_shared_down, W_gate,
           W_down, W_pgate):
    B, S, D = hidden_states.shape
    E = W_router.shape[1]
    DS = W_shared_gate.shape[1]
    T = B * S
    K = 2
    TK = T * K

    BT = 256 if TK >= 2048 else 8
    RB = 512 if T >= 2048 else T
    TB = 512 if T >= 2048 else T
    DSB = 512 if DS >= 2048 else DS
    NB = TK // BT + E
    NMAX = NB * BT

    h = hidden_states.reshape(T, D)
    logits, e2, w2, rank, counts, aux = _router(h, W_router, RB)
    pflat, blk_e = _meta(counts, e2, rank, BT, NB)

    hb = h.astype(jnp.bfloat16)
    y = _shared(hb, W_shared_gate.astype(jnp.bfloat16),
                W_shared_down.astype(jnp.bfloat16), e2,
                W_pgate.T.astype(jnp.bfloat16), TB, DSB)

    pflat_f = pflat.reshape(TK)
    x_d = _dispatch_scatter(y.reshape(TK, D), pflat_f, NMAX)
    out_d = _expert_ffn(blk_e.reshape(NB), x_d, W_gate, W_down, BT)
    final = _combine(out_d, pflat_f, w2, T)

    return final.reshape(B, S, D), logits, aux[0, 0]


# speedup vs baseline: 7.8103x; 7.8103x over previous
import functools


def _pack_rows(vals_bf16):
    n = vals_bf16.shape[-1] // 2
    lo = lax.bitcast_convert_type(vals_bf16[..., :n],
                                  jnp.uint16).astype(jnp.uint32)
    hi = lax.bitcast_convert_type(vals_bf16[..., n:],
                                  jnp.uint16).astype(jnp.uint32)
    return lax.bitcast_convert_type(lo | (hi << 16), jnp.int32)


def _unpack_rows(packed_i32):
    u = lax.bitcast_convert_type(packed_i32, jnp.uint32)
    lo = lax.bitcast_convert_type((u & 0xFFFF).astype(jnp.uint16),
                                  jnp.bfloat16)
    hi = lax.bitcast_convert_type((u >> 16).astype(jnp.uint16),
                                  jnp.bfloat16)
    return jnp.concatenate([lo, hi], axis=-1)

import jax
import jax.numpy as jnp
from jax import lax
from jax.experimental import pallas as pl
from jax.experimental.pallas import tpu as pltpu
from jax.experimental.pallas import tpu_sc as plsc


def _router_body(E, RB, h_ref, wr_ref, logits_ref, e2_ref, w2_ref, rank_ref,
                 cnt_ref, aux_ref, runcnt, impacc):
    i = pl.program_id(0)
    nsteps = pl.num_programs(0)

    @pl.when(i == 0)
    def _():
        runcnt[...] = jnp.zeros_like(runcnt)
        impacc[...] = jnp.zeros_like(impacc)

    h = h_ref[...].astype(jnp.bfloat16)
    logits = lax.dot_general(h, wr_ref[...].astype(jnp.bfloat16),
                             (((1,), (0,)), ((), ())),
                             preferred_element_type=jnp.float32)
    logits_ref[...] = logits

    m = jnp.max(logits, axis=1, keepdims=True)
    ex = jnp.exp(logits - m)
    probs = ex / jnp.sum(ex, axis=1, keepdims=True)

    iota_e = lax.broadcasted_iota(jnp.int32, probs.shape, 1)
    m1 = jnp.max(probs, axis=1, keepdims=True)
    a1 = jnp.min(jnp.where(probs == m1, iota_e, E), axis=1, keepdims=True)
    masked = jnp.where(iota_e == a1, -1.0, probs)
    m2 = jnp.max(masked, axis=1, keepdims=True)
    a2 = jnp.min(jnp.where(masked == m2, iota_e, E), axis=1, keepdims=True)

    e2 = jnp.concatenate([a1, a2], axis=1)
    w2 = jnp.concatenate([m1, m2], axis=1)
    e2_ref[...] = e2
    w2_ref[...] = w2

    iota_e3 = lax.broadcasted_iota(jnp.int32, (1, 1, E), 2)
    oh = (e2[:, :, None] == iota_e3).astype(jnp.float32)
    oh_f = oh.reshape(RB * 2, E)
    n = RB * 2
    tri = (lax.broadcasted_iota(jnp.int32, (n, n), 0)
           >= lax.broadcasted_iota(jnp.int32, (n, n), 1)).astype(jnp.bfloat16)
    c = lax.dot_general(tri, oh_f.astype(jnp.bfloat16),
                        (((1,), (0,)), ((), ())),
                        preferred_element_type=jnp.float32)
    c = c.astype(jnp.int32)
    oh_i = oh_f.astype(jnp.int32)
    rank_f = jnp.sum(oh_i * (runcnt[...] + c - 1), axis=1)
    rank_ref[...] = rank_f.reshape(RB, 2)

    runcnt[...] = runcnt[...] + c[-1:, :]
    impacc[...] = impacc[...] + jnp.sum(probs, axis=0, keepdims=True)

    @pl.when(i == nsteps - 1)
    def _():
        cnt = runcnt[...]
        cnt_ref[...] = cnt
        eps = 1e-09
        imp = impacc[...]
        impn = imp / (jnp.sum(imp) + eps)
        cntf = cnt.astype(jnp.float32)
        ldn = cntf / (jnp.sum(cntf) + eps)
        aux_ref[...] = jnp.full((1, 1), E, jnp.float32) * jnp.sum(impn * ldn)


def _router(h, W_router, RB):
    T, D = h.shape
    E = W_router.shape[1]
    grid = (T // RB,)
    return pl.pallas_call(
        functools.partial(_router_body, E, RB),
        grid=grid,
        in_specs=[
            pl.BlockSpec((RB, D), lambda i: (i, 0)),
            pl.BlockSpec((D, E), lambda i: (0, 0)),
        ],
        out_specs=[
            pl.BlockSpec((RB, E), lambda i: (i, 0)),
            pl.BlockSpec((RB, 2), lambda i: (i, 0)),
            pl.BlockSpec((RB, 2), lambda i: (i, 0)),
            pl.BlockSpec((RB, 2), lambda i: (i, 0)),
            pl.BlockSpec((1, E), lambda i: (0, 0)),
            pl.BlockSpec((1, 1), lambda i: (0, 0)),
        ],
        out_shape=[
            jax.ShapeDtypeStruct((T, E), jnp.float32),
            jax.ShapeDtypeStruct((T, 2), jnp.int32),
            jax.ShapeDtypeStruct((T, 2), jnp.float32),
            jax.ShapeDtypeStruct((T, 2), jnp.int32),
            jax.ShapeDtypeStruct((1, E), jnp.int32),
            jax.ShapeDtypeStruct((1, 1), jnp.float32),
        ],
        scratch_shapes=[
            pltpu.VMEM((1, E), jnp.int32),
            pltpu.VMEM((1, E), jnp.float32),
        ],
        compiler_params=pltpu.CompilerParams(
            dimension_semantics=("arbitrary",)),
    )(h, W_router)


def _meta_body(E, BT, NB, cnt_ref, e2_ref, rank_ref, pflat_ref, blk_ref):
    cnt = cnt_ref[...]
    padded = ((cnt + BT - 1) // BT) * BT
    ii = lax.broadcasted_iota(jnp.int32, (1, E, E), 1)
    jj = lax.broadcasted_iota(jnp.int32, (1, E, E), 2)
    csum = jnp.sum(jnp.where(jj <= ii, padded[:, None, :], 0), axis=2)
    start = csum - padded

    e2 = e2_ref[...]
    iota_e = lax.broadcasted_iota(jnp.int32, (1, 1, E), 2)
    oh = (e2[:, :, None] == iota_e).astype(jnp.int32)
    st = jnp.sum(oh * start[:, None, :], axis=2)
    pflat_ref[...] = st + rank_ref[...]

    bidx = lax.broadcasted_iota(jnp.int32, (1, NB), 1) * BT
    ge = (bidx[:, :, None] >= csum[:, None, :]).astype(jnp.int32)
    blk_ref[...] = jnp.minimum(jnp.sum(ge, axis=2), E - 1)


def _meta(cnt, e2, rank, BT, NB):
    T = e2.shape[0]
    E = cnt.shape[1]
    return pl.pallas_call(
        functools.partial(_meta_body, E, BT, NB),
        out_shape=[
            jax.ShapeDtypeStruct((T, 2), jnp.int32),
            jax.ShapeDtypeStruct((1, NB), jnp.int32),
        ],
    )(cnt, e2, rank)


def _shared_body(E, NJ, hb_ref, wsg_ref, wsd_ref, e2_ref, wpgt_ref, y_ref,
                 acc):
    j = pl.program_id(1)

    @pl.when(j == 0)
    def _():
        acc[...] = jnp.zeros_like(acc)

    a = lax.dot_general(hb_ref[...], wsg_ref[...], (((1,), (0,)), ((), ())),
                        preferred_element_type=jnp.float32)
    a = jnp.maximum(a, 0.0).astype(jnp.bfloat16)
    acc[...] = acc[...] + lax.dot_general(
        a, wsd_ref[...], (((1,), (0,)), ((), ())),
        preferred_element_type=jnp.float32)

    @pl.when(j == NJ - 1)
    def _():
        hs = acc[...]
        p = lax.dot_general(hs.astype(jnp.bfloat16), wpgt_ref[...],
                            (((1,), (0,)), ((), ())),
                            preferred_element_type=jnp.float32)
        p = jax.nn.sigmoid(p)
        e2 = e2_ref[...]
        iota_e2 = lax.broadcasted_iota(jnp.int32, p.shape, 1)
        for k in range(2):
            psel = jnp.sum(
                jnp.where(e2[:, k:k + 1] == iota_e2, p, 0.0),
                axis=1, keepdims=True)
            y_ref[:, k, :] = _pack_rows((psel * hs).astype(jnp.bfloat16))


def _shared(hb, Wsg, Wsd, e2, Wpgt, TB, DSB):
    T, D = hb.shape
    DS = Wsg.shape[1]
    E = Wpgt.shape[1]
    NJ = DS // DSB
    grid = (T // TB, NJ)
    return pl.pallas_call(
        functools.partial(_shared_body, E, NJ),
        grid=grid,
        in_specs=[
            pl.BlockSpec((TB, D), lambda i, j: (i, 0)),
            pl.BlockSpec((D, DSB), lambda i, j: (0, j)),
            pl.BlockSpec((DSB, D), lambda i, j: (j, 0)),
            pl.BlockSpec((TB, 2), lambda i, j: (i, 0)),
            pl.BlockSpec((D, E), lambda i, j: (0, 0)),
        ],
        out_specs=pl.BlockSpec((TB, 2, D // 2), lambda i, j: (i, 0, 0)),
        out_shape=jax.ShapeDtypeStruct((T, 2, D // 2), jnp.int32),
        scratch_shapes=[pltpu.VMEM((TB, D), jnp.float32)],
        compiler_params=pltpu.CompilerParams(
            dimension_semantics=("parallel", "arbitrary")),
    )(hb, Wsg, Wsd, e2, Wpgt)


_SC_CH = 32


def _sc_dispatch(src, idx3, out_rows, gather):
    D2 = src.shape[1]
    SL = D2 // 128
    NW, NCHUNK, CH = idx3.shape
    src3 = src.reshape(src.shape[0], SL, 128)
    mesh = plsc.VectorSubcoreMesh(core_axis_name="c", subcore_axis_name="s")

    @functools.partial(
        pl.kernel, mesh=mesh,
        out_type=jax.ShapeDtypeStruct((out_rows, SL, 128), jnp.int32),
        scratch_types=[
            pltpu.VMEM((NCHUNK, CH), jnp.int32),
            pltpu.VMEM((CH, SL, 128), jnp.int32),
            pltpu.SemaphoreType.DMA,
        ],
    )
    def k(src_hbm, idx_hbm, out_hbm, idx_v, rows_v, sem):
        wid = lax.axis_index("s") * 2 + lax.axis_index("c")
        pltpu.sync_copy(idx_hbm.at[wid], idx_v)
        base = wid * (NCHUNK * CH)
        for j in range(NCHUNK):
            if gather:
                pltpu.async_copy(src_hbm.at[idx_v.at[j]], rows_v, sem).wait()
                pltpu.sync_copy(rows_v, out_hbm.at[pl.ds(base + j * CH, CH)])
            else:
                pltpu.sync_copy(src_hbm.at[pl.ds(base + j * CH, CH)], rows_v)
                pltpu.async_copy(rows_v, out_hbm.at[idx_v.at[j]], sem).wait()

    out = k(src3, idx3)
    return out.reshape(out_rows, D2)


def _expert_body(be_ref, x_ref, wg_ref, wd_ref, o_ref):
    x = _unpack_rows(x_ref[...])
    wg = wg_ref[0].astype(jnp.bfloat16)
    t = lax.dot_general(x, wg, (((1,), (0,)), ((), ())),
                        preferred_element_type=jnp.float32)
    t = jnp.maximum(t, 0.0).astype(jnp.bfloat16)
    o = lax.dot_general(t, wd_ref[0].astype(jnp.bfloat16),
                        (((1,), (0,)), ((), ())),
                        preferred_element_type=jnp.float32)
    o_ref[...] = _pack_rows(o.astype(jnp.bfloat16))


def _expert_ffn(blk_e, x_d, W_gate, W_down, BT):
    NMAX, D2 = x_d.shape
    E, D, DP = W_gate.shape
    NB = NMAX // BT
    grid_spec = pltpu.PrefetchScalarGridSpec(
        num_scalar_prefetch=1,
        grid=(NB,),
        in_specs=[
            pl.BlockSpec((BT, D2), lambda i, be: (i, 0)),
            pl.BlockSpec((1, D, DP), lambda i, be: (be[i], 0, 0)),
            pl.BlockSpec((1, DP, D), lambda i, be: (be[i], 0, 0)),
        ],
        out_specs=pl.BlockSpec((BT, D2), lambda i, be: (i, 0)),
    )
    return pl.pallas_call(
        _expert_body,
        grid_spec=grid_spec,
        out_shape=jax.ShapeDtypeStruct((NMAX, D2), jnp.int32),
        compiler_params=pltpu.CompilerParams(
            dimension_semantics=("arbitrary",)),
    )(blk_e, x_d, W_gate, W_down)


def _combine_body(c_ref, w_ref, o_ref):
    w = w_ref[...]
    acc = w[:, 0:1] * _unpack_rows(c_ref[:, 0, :]).astype(jnp.float32)
    acc = acc + w[:, 1:2] * _unpack_rows(c_ref[:, 1, :]).astype(jnp.float32)
    o_ref[...] = acc


def _combine(comb, w2, TB):
    T, K, D2 = comb.shape
    return pl.pallas_call(
        _combine_body,
        grid=(T // TB,),
        in_specs=[
            pl.BlockSpec((TB, K, D2), lambda i: (i, 0, 0)),
            pl.BlockSpec((TB, K), lambda i: (i, 0)),
        ],
        out_specs=pl.BlockSpec((TB, D2 * 2), lambda i: (i, 0)),
        out_shape=jax.ShapeDtypeStruct((T, D2 * 2), jnp.float32),
        compiler_params=pltpu.CompilerParams(
            dimension_semantics=("parallel",)),
    )(comb, w2)


def kernel(hidden_states, W_router, W_shared_gate, W_shared_down, W_gate,
           W_down, W_pgate):
    B, S, D = hidden_states.shape
    E = W_router.shape[1]
    DS = W_shared_gate.shape[1]
    T = B * S
    K = 2
    TK = T * K

    BT = 256 if TK >= 2048 else 8
    RB = 512 if T >= 2048 else T
    TB = 512 if T >= 2048 else T
    DSB = 512 if DS >= 2048 else DS
    NB = TK // BT + E
    NMAX = NB * BT

    h = hidden_states.reshape(T, D)
    logits, e2, w2, rank, counts, aux = _router(h, W_router, RB)
    pflat, blk_e = _meta(counts, e2, rank, BT, NB)

    hb = h.astype(jnp.bfloat16)
    y = _shared(hb, W_shared_gate.astype(jnp.bfloat16),
                W_shared_down.astype(jnp.bfloat16), e2,
                W_pgate.T.astype(jnp.bfloat16), TB, DSB)

    NW, CH = 32, _SC_CH
    D2 = D // 2
    idx3 = pflat.reshape(NW, TK // (NW * CH), CH)
    x_d = _sc_dispatch(y.reshape(TK, D2), idx3, NMAX, gather=False)
    out_d = _expert_ffn(blk_e.reshape(NB), x_d, W_gate, W_down, BT)
    comb = _sc_dispatch(out_d, idx3, TK, gather=True)
    final = _combine(comb.reshape(T, K, D2), w2, TB)

    return final.reshape(B, S, D), logits, aux[0, 0]


# speedup vs baseline: 8.4893x; 1.0869x over previous
import functools


def _pack_rows(vals_bf16):
    n = vals_bf16.shape[-1] // 2
    lo = lax.bitcast_convert_type(vals_bf16[..., :n],
                                  jnp.uint16).astype(jnp.uint32)
    hi = lax.bitcast_convert_type(vals_bf16[..., n:],
                                  jnp.uint16).astype(jnp.uint32)
    return lax.bitcast_convert_type(lo | (hi << 16), jnp.int32)


def _unpack_rows(packed_i32):
    u = lax.bitcast_convert_type(packed_i32, jnp.uint32)
    lo = lax.bitcast_convert_type((u & 0xFFFF).astype(jnp.uint16),
                                  jnp.bfloat16)
    hi = lax.bitcast_convert_type((u >> 16).astype(jnp.uint16),
                                  jnp.bfloat16)
    return jnp.concatenate([lo, hi], axis=-1)

import jax
import jax.numpy as jnp
from jax import lax
from jax.experimental import pallas as pl
from jax.experimental.pallas import tpu as pltpu
from jax.experimental.pallas import tpu_sc as plsc


def _router_body(E, RB, h_ref, wr_ref, logits_ref, e2_ref, w2_ref, rank_ref,
                 cnt_ref, aux_ref, runcnt, impacc):
    i = pl.program_id(0)
    nsteps = pl.num_programs(0)

    @pl.when(i == 0)
    def _():
        runcnt[...] = jnp.zeros_like(runcnt)
        impacc[...] = jnp.zeros_like(impacc)

    logits = lax.dot_general(h_ref[...], wr_ref[...],
                             (((1,), (0,)), ((), ())),
                             preferred_element_type=jnp.float32)
    logits_ref[...] = logits

    m = jnp.max(logits, axis=1, keepdims=True)
    ex = jnp.exp(logits - m)
    probs = ex / jnp.sum(ex, axis=1, keepdims=True)

    iota_e = lax.broadcasted_iota(jnp.int32, probs.shape, 1)
    m1 = jnp.max(probs, axis=1, keepdims=True)
    a1 = jnp.min(jnp.where(probs == m1, iota_e, E), axis=1, keepdims=True)
    masked = jnp.where(iota_e == a1, -1.0, probs)
    m2 = jnp.max(masked, axis=1, keepdims=True)
    a2 = jnp.min(jnp.where(masked == m2, iota_e, E), axis=1, keepdims=True)

    e2 = jnp.concatenate([a1, a2], axis=1)
    w2 = jnp.concatenate([m1, m2], axis=1)
    e2_ref[...] = e2
    w2_ref[...] = w2

    iota_e3 = lax.broadcasted_iota(jnp.int32, (1, 1, E), 2)
    oh = (e2[:, :, None] == iota_e3).astype(jnp.float32)
    oh_f = oh.reshape(RB * 2, E)
    n = RB * 2
    tri = (lax.broadcasted_iota(jnp.int32, (n, n), 0)
           >= lax.broadcasted_iota(jnp.int32, (n, n), 1)).astype(jnp.bfloat16)
    c = lax.dot_general(tri, oh_f.astype(jnp.bfloat16),
                        (((1,), (0,)), ((), ())),
                        preferred_element_type=jnp.float32)
    c = c.astype(jnp.int32)
    oh_i = oh_f.astype(jnp.int32)
    rank_f = jnp.sum(oh_i * (runcnt[...] + c - 1), axis=1)
    rank_ref[...] = rank_f.reshape(RB, 2)

    runcnt[...] = runcnt[...] + c[-1:, :]
    impacc[...] = impacc[...] + jnp.sum(probs, axis=0, keepdims=True)

    @pl.when(i == nsteps - 1)
    def _():
        cnt = runcnt[...]
        cnt_ref[...] = cnt
        eps = 1e-09
        imp = impacc[...]
        impn = imp / (jnp.sum(imp) + eps)
        cntf = cnt.astype(jnp.float32)
        ldn = cntf / (jnp.sum(cntf) + eps)
        aux_ref[...] = jnp.full((1, 1), E, jnp.float32) * jnp.sum(impn * ldn)


def _router(h, W_router, RB):
    T, D = h.shape
    E = W_router.shape[1]
    grid = (T // RB,)
    return pl.pallas_call(
        functools.partial(_router_body, E, RB),
        grid=grid,
        in_specs=[
            pl.BlockSpec((RB, D), lambda i: (i, 0)),
            pl.BlockSpec((D, E), lambda i: (0, 0)),
        ],
        out_specs=[
            pl.BlockSpec((RB, E), lambda i: (i, 0)),
            pl.BlockSpec((RB, 2), lambda i: (i, 0)),
            pl.BlockSpec((RB, 2), lambda i: (i, 0)),
            pl.BlockSpec((RB, 2), lambda i: (i, 0)),
            pl.BlockSpec((1, E), lambda i: (0, 0)),
            pl.BlockSpec((1, 1), lambda i: (0, 0)),
        ],
        out_shape=[
            jax.ShapeDtypeStruct((T, E), jnp.float32),
            jax.ShapeDtypeStruct((T, 2), jnp.int32),
            jax.ShapeDtypeStruct((T, 2), jnp.float32),
            jax.ShapeDtypeStruct((T, 2), jnp.int32),
            jax.ShapeDtypeStruct((1, E), jnp.int32),
            jax.ShapeDtypeStruct((1, 1), jnp.float32),
        ],
        scratch_shapes=[
            pltpu.VMEM((1, E), jnp.int32),
            pltpu.VMEM((1, E), jnp.float32),
        ],
        compiler_params=pltpu.CompilerParams(
            dimension_semantics=("arbitrary",)),
    )(h, W_router)


def _meta_body(E, BT, NB, cnt_ref, e2_ref, rank_ref, pflat_ref, blk_ref):
    cnt = cnt_ref[...]
    padded = ((cnt + BT - 1) // BT) * BT
    ii = lax.broadcasted_iota(jnp.int32, (1, E, E), 1)
    jj = lax.broadcasted_iota(jnp.int32, (1, E, E), 2)
    csum = jnp.sum(jnp.where(jj <= ii, padded[:, None, :], 0), axis=2)
    start = csum - padded

    e2 = e2_ref[...]
    iota_e = lax.broadcasted_iota(jnp.int32, (1, 1, E), 2)
    oh = (e2[:, :, None] == iota_e).astype(jnp.int32)
    st = jnp.sum(oh * start[:, None, :], axis=2)
    pflat_ref[...] = st + rank_ref[...]

    bidx = lax.broadcasted_iota(jnp.int32, (1, NB), 1) * BT
    ge = (bidx[:, :, None] >= csum[:, None, :]).astype(jnp.int32)
    blk_ref[...] = jnp.minimum(jnp.sum(ge, axis=2), E - 1)


def _meta(cnt, e2, rank, BT, NB):
    T = e2.shape[0]
    E = cnt.shape[1]
    return pl.pallas_call(
        functools.partial(_meta_body, E, BT, NB),
        out_shape=[
            jax.ShapeDtypeStruct((T, 2), jnp.int32),
            jax.ShapeDtypeStruct((1, NB), jnp.int32),
        ],
    )(cnt, e2, rank)


def _shared_body(E, NJ, hb_ref, wsg_ref, wsd_ref, e2_ref, w2_ref, wpgt_ref,
                 y_ref, acc):
    j = pl.program_id(1)

    @pl.when(j == 0)
    def _():
        acc[...] = jnp.zeros_like(acc)

    a = lax.dot_general(hb_ref[...], wsg_ref[...], (((1,), (0,)), ((), ())),
                        preferred_element_type=jnp.float32)
    a = jnp.maximum(a, 0.0).astype(jnp.bfloat16)
    acc[...] = acc[...] + lax.dot_general(
        a, wsd_ref[...], (((1,), (0,)), ((), ())),
        preferred_element_type=jnp.float32)

    @pl.when(j == NJ - 1)
    def _():
        hs = acc[...]
        p = lax.dot_general(hs.astype(jnp.bfloat16), wpgt_ref[...],
                            (((1,), (0,)), ((), ())),
                            preferred_element_type=jnp.float32)
        p = jax.nn.sigmoid(p)
        e2 = e2_ref[...]
        iota_e2 = lax.broadcasted_iota(jnp.int32, p.shape, 1)
        for k in range(2):
            psel = jnp.sum(
                jnp.where(e2[:, k:k + 1] == iota_e2, p, 0.0),
                axis=1, keepdims=True)
            psel = psel * w2_ref[:, k:k + 1]
            y_ref[:, k, :] = _pack_rows((psel * hs).astype(jnp.bfloat16))


def _shared(hb, Wsg, Wsd, e2, w2, Wpgt, TB, DSB):
    T, D = hb.shape
    DS = Wsg.shape[1]
    E = Wpgt.shape[1]
    NJ = DS // DSB
    grid = (T // TB, NJ)
    return pl.pallas_call(
        functools.partial(_shared_body, E, NJ),
        grid=grid,
        in_specs=[
            pl.BlockSpec((TB, D), lambda i, j: (i, 0)),
            pl.BlockSpec((D, DSB), lambda i, j: (0, j)),
            pl.BlockSpec((DSB, D), lambda i, j: (j, 0)),
            pl.BlockSpec((TB, 2), lambda i, j: (i, 0)),
            pl.BlockSpec((TB, 2), lambda i, j: (i, 0)),
            pl.BlockSpec((D, E), lambda i, j: (0, 0)),
        ],
        out_specs=pl.BlockSpec((TB, 2, D // 2), lambda i, j: (i, 0, 0)),
        out_shape=jax.ShapeDtypeStruct((T, 2, D // 2), jnp.int32),
        scratch_shapes=[pltpu.VMEM((TB, D), jnp.float32)],
        compiler_params=pltpu.CompilerParams(
            dimension_semantics=("parallel", "arbitrary")),
    )(hb, Wsg, Wsd, e2, w2, Wpgt)


_SC_CH = 32


def _sc_dispatch(src, idx3, out_rows, gather):
    D2 = src.shape[1]
    SL = D2 // 128
    NW, NCHUNK, CH = idx3.shape
    src3 = src.reshape(src.shape[0], SL, 128)
    mesh = plsc.VectorSubcoreMesh(core_axis_name="c", subcore_axis_name="s")

    @functools.partial(
        pl.kernel, mesh=mesh,
        out_type=jax.ShapeDtypeStruct((out_rows, SL, 128), jnp.int32),
        scratch_types=[
            pltpu.VMEM((NCHUNK, CH), jnp.int32),
            pltpu.VMEM((CH, SL, 128), jnp.int32),
            pltpu.VMEM((CH, SL, 128), jnp.int32),
            pltpu.SemaphoreType.DMA,
            pltpu.SemaphoreType.DMA,
            pltpu.SemaphoreType.DMA,
            pltpu.SemaphoreType.DMA,
        ],
    )
    def k(src_hbm, idx_hbm, out_hbm, idx_v, rows0, rows1,
          rs0, rs1, ws0, ws1):
        wid = lax.axis_index("s") * 2 + lax.axis_index("c")
        pltpu.sync_copy(idx_hbm.at[wid], idx_v)
        base = wid * (NCHUNK * CH)
        rows = (rows0, rows1)
        rsem = (rs0, rs1)
        wsem = (ws0, ws1)

        def rd(j):
            b = j % 2
            if gather:
                return pltpu.make_async_copy(
                    src_hbm.at[idx_v.at[j]], rows[b], rsem[b])
            return pltpu.make_async_copy(
                src_hbm.at[pl.ds(base + j * CH, CH)], rows[b], rsem[b])

        def wr(j):
            b = j % 2
            if gather:
                return pltpu.make_async_copy(
                    rows[b], out_hbm.at[pl.ds(base + j * CH, CH)], wsem[b])
            return pltpu.make_async_copy(
                rows[b], out_hbm.at[idx_v.at[j]], wsem[b])

        rd(0).start()
        for j in range(NCHUNK):
            if j + 1 < NCHUNK:
                if j >= 1:
                    wr(j - 1).wait()
                rd(j + 1).start()
            rd(j).wait()
            wr(j).start()
        wr(NCHUNK - 2).wait()
        wr(NCHUNK - 1).wait()

    out = k(src3, idx3)
    return out.reshape(out_rows, D2)


def _expert_body(be_ref, x_ref, wg_ref, wd_ref, o_ref):
    x = _unpack_rows(x_ref[...])
    wg = wg_ref[0].astype(jnp.bfloat16)
    t = lax.dot_general(x, wg, (((1,), (0,)), ((), ())),
                        preferred_element_type=jnp.float32)
    t = jnp.maximum(t, 0.0).astype(jnp.bfloat16)
    o = lax.dot_general(t, wd_ref[0].astype(jnp.bfloat16),
                        (((1,), (0,)), ((), ())),
                        preferred_element_type=jnp.float32)
    o_ref[...] = _pack_rows(o.astype(jnp.bfloat16))


def _expert_ffn(blk_e, x_d, W_gate, W_down, BT):
    NMAX, D2 = x_d.shape
    E, D, DP = W_gate.shape
    NB = NMAX // BT
    grid_spec = pltpu.PrefetchScalarGridSpec(
        num_scalar_prefetch=1,
        grid=(NB,),
        in_specs=[
            pl.BlockSpec((BT, D2), lambda i, be: (i, 0)),
            pl.BlockSpec((1, D, DP), lambda i, be: (be[i], 0, 0)),
            pl.BlockSpec((1, DP, D), lambda i, be: (be[i], 0, 0)),
        ],
        out_specs=pl.BlockSpec((BT, D2), lambda i, be: (i, 0)),
    )
    return pl.pallas_call(
        _expert_body,
        grid_spec=grid_spec,
        out_shape=jax.ShapeDtypeStruct((NMAX, D2), jnp.int32),
        compiler_params=pltpu.CompilerParams(
            dimension_semantics=("arbitrary",)),
    )(blk_e, x_d, W_gate, W_down)


def _combine_body(c_ref, o_ref):
    D2 = c_ref.shape[2]
    u0 = lax.bitcast_convert_type(c_ref[:, 0, :], jnp.uint32)
    u1 = lax.bitcast_convert_type(c_ref[:, 1, :], jnp.uint32)

    def half(u):
        return lax.bitcast_convert_type((u & 0xFFFF).astype(jnp.uint16),
                                        jnp.bfloat16).astype(jnp.float32)

    def half_hi(u):
        return lax.bitcast_convert_type((u >> 16).astype(jnp.uint16),
                                        jnp.bfloat16).astype(jnp.float32)

    o_ref[:, :D2] = half(u0) + half(u1)
    o_ref[:, D2:] = half_hi(u0) + half_hi(u1)


def _combine(comb, TB):
    T, K, D2 = comb.shape
    return pl.pallas_call(
        _combine_body,
        grid=(T // TB,),
        in_specs=[
            pl.BlockSpec((TB, K, D2), lambda i: (i, 0, 0)),
        ],
        out_specs=pl.BlockSpec((TB, D2 * 2), lambda i: (i, 0)),
        out_shape=jax.ShapeDtypeStruct((T, D2 * 2), jnp.float32),
        compiler_params=pltpu.CompilerParams(
            dimension_semantics=("parallel",)),
    )(comb)


def kernel(hidden_states, W_router, W_shared_gate, W_shared_down, W_gate,
           W_down, W_pgate):
    B, S, D = hidden_states.shape
    E = W_router.shape[1]
    DS = W_shared_gate.shape[1]
    T = B * S
    K = 2
    TK = T * K

    BT = 256 if TK >= 2048 else 8
    RB = 512 if T >= 2048 else T
    TB = 512 if T >= 2048 else T
    DSB = 512 if DS >= 2048 else DS
    NB = TK // BT + E
    NMAX = NB * BT

    h = hidden_states.reshape(T, D)
    hb = h.astype(jnp.bfloat16)
    logits, e2, w2, rank, counts, aux = _router(
        hb, W_router.astype(jnp.bfloat16), RB)
    pflat, blk_e = _meta(counts, e2, rank, BT, NB)

    y = _shared(hb, W_shared_gate.astype(jnp.bfloat16),
                W_shared_down.astype(jnp.bfloat16), e2, w2,
                W_pgate.T.astype(jnp.bfloat16), TB, DSB)

    NW, CH = 32, _SC_CH
    D2 = D // 2
    idx3 = pflat.reshape(NW, TK // (NW * CH), CH)
    x_d = _sc_dispatch(y.reshape(TK, D2), idx3, NMAX, gather=False)
    out_d = _expert_ffn(blk_e.reshape(NB), x_d, W_gate, W_down, BT)
    comb = _sc_dispatch(out_d, idx3, TK, gather=True)
    final = _combine(comb.reshape(T, K, D2), TB)

    return final.reshape(B, S, D), logits, aux[0, 0]


# speedup vs baseline: 8.8573x; 1.0434x over previous
import functools


def _pack_rows(vals_bf16):
    n = vals_bf16.shape[-1] // 2
    lo = lax.bitcast_convert_type(vals_bf16[..., :n],
                                  jnp.uint16).astype(jnp.uint32)
    hi = lax.bitcast_convert_type(vals_bf16[..., n:],
                                  jnp.uint16).astype(jnp.uint32)
    return lax.bitcast_convert_type(lo | (hi << 16), jnp.int32)


def _unpack_rows(packed_i32):
    u = lax.bitcast_convert_type(packed_i32, jnp.uint32)
    lo = lax.bitcast_convert_type((u & 0xFFFF).astype(jnp.uint16),
                                  jnp.bfloat16)
    hi = lax.bitcast_convert_type((u >> 16).astype(jnp.uint16),
                                  jnp.bfloat16)
    return jnp.concatenate([lo, hi], axis=-1)

import jax
import jax.numpy as jnp
from jax import lax
from jax.experimental import pallas as pl
from jax.experimental.pallas import tpu as pltpu
from jax.experimental.pallas import tpu_sc as plsc


def _router_body(E, RB, BT, NB, h_ref, wr_ref, logits_ref, e2_ref, w2_ref,
                 pflat_ref, blk_ref, aux_ref, runcnt, impacc, e2s, rnks):
    i = pl.program_id(0)
    nsteps = pl.num_programs(0)

    @pl.when(i == 0)
    def _():
        runcnt[...] = jnp.zeros_like(runcnt)
        impacc[...] = jnp.zeros_like(impacc)

    logits = lax.dot_general(h_ref[...], wr_ref[...],
                             (((1,), (0,)), ((), ())),
                             preferred_element_type=jnp.float32)
    logits_ref[...] = logits

    m = jnp.max(logits, axis=1, keepdims=True)
    ex = jnp.exp(logits - m)
    probs = ex / jnp.sum(ex, axis=1, keepdims=True)

    iota_e = lax.broadcasted_iota(jnp.int32, probs.shape, 1)
    m1 = jnp.max(probs, axis=1, keepdims=True)
    a1 = jnp.min(jnp.where(probs == m1, iota_e, E), axis=1, keepdims=True)
    masked = jnp.where(iota_e == a1, -1.0, probs)
    m2 = jnp.max(masked, axis=1, keepdims=True)
    a2 = jnp.min(jnp.where(masked == m2, iota_e, E), axis=1, keepdims=True)

    e2 = jnp.concatenate([a1, a2], axis=1)
    w2 = jnp.concatenate([m1, m2], axis=1)
    e2_ref[...] = e2
    w2_ref[...] = w2

    iota_e3 = lax.broadcasted_iota(jnp.int32, (1, 1, E), 2)
    oh = (e2[:, :, None] == iota_e3).astype(jnp.float32)
    oh_f = oh.reshape(RB * 2, E)
    n = RB * 2
    tri = (lax.broadcasted_iota(jnp.int32, (n, n), 0)
           >= lax.broadcasted_iota(jnp.int32, (n, n), 1)).astype(jnp.bfloat16)
    c = lax.dot_general(tri, oh_f.astype(jnp.bfloat16),
                        (((1,), (0,)), ((), ())),
                        preferred_element_type=jnp.float32)
    c = c.astype(jnp.int32)
    oh_i = oh_f.astype(jnp.int32)
    rank_f = jnp.sum(oh_i * (runcnt[...] + c - 1), axis=1)
    e2s[pl.ds(i * RB, RB), :] = e2
    rnks[pl.ds(i * RB, RB), :] = rank_f.reshape(RB, 2)

    runcnt[...] = runcnt[...] + c[-1:, :]
    impacc[...] = impacc[...] + jnp.sum(probs, axis=0, keepdims=True)

    @pl.when(i == nsteps - 1)
    def _():
        cnt = runcnt[...]
        eps = 1e-09
        imp = impacc[...]
        impn = imp / (jnp.sum(imp) + eps)
        cntf = cnt.astype(jnp.float32)
        ldn = cntf / (jnp.sum(cntf) + eps)
        aux_ref[...] = jnp.full((1, 1), E, jnp.float32) * jnp.sum(impn * ldn)

        padded = ((cnt + BT - 1) // BT) * BT
        ii = lax.broadcasted_iota(jnp.int32, (1, E, E), 1)
        jj = lax.broadcasted_iota(jnp.int32, (1, E, E), 2)
        csum = jnp.sum(jnp.where(jj <= ii, padded[:, None, :], 0), axis=2)
        start = csum - padded

        e2a = e2s[...]
        iota3 = lax.broadcasted_iota(jnp.int32, (1, 1, E), 2)
        oha = (e2a[:, :, None] == iota3).astype(jnp.int32)
        st = jnp.sum(oha * start[:, None, :], axis=2)
        pflat_ref[...] = st + rnks[...]

        bidx = lax.broadcasted_iota(jnp.int32, (1, NB), 1) * BT
        ge = (bidx[:, :, None] >= csum[:, None, :]).astype(jnp.int32)
        blk_ref[...] = jnp.minimum(jnp.sum(ge, axis=2), E - 1)


def _router(h, W_router, RB, BT, NB):
    T, D = h.shape
    E = W_router.shape[1]
    grid = (T // RB,)
    return pl.pallas_call(
        functools.partial(_router_body, E, RB, BT, NB),
        grid=grid,
        in_specs=[
            pl.BlockSpec((RB, D), lambda i: (i, 0)),
            pl.BlockSpec((D, E), lambda i: (0, 0)),
        ],
        out_specs=[
            pl.BlockSpec((RB, E), lambda i: (i, 0)),
            pl.BlockSpec((RB, 2), lambda i: (i, 0)),
            pl.BlockSpec((RB, 2), lambda i: (i, 0)),
            pl.BlockSpec((T, 2), lambda i: (0, 0)),
            pl.BlockSpec((1, NB), lambda i: (0, 0)),
            pl.BlockSpec((1, 1), lambda i: (0, 0)),
        ],
        out_shape=[
            jax.ShapeDtypeStruct((T, E), jnp.float32),
            jax.ShapeDtypeStruct((T, 2), jnp.int32),
            jax.ShapeDtypeStruct((T, 2), jnp.float32),
            jax.ShapeDtypeStruct((T, 2), jnp.int32),
            jax.ShapeDtypeStruct((1, NB), jnp.int32),
            jax.ShapeDtypeStruct((1, 1), jnp.float32),
        ],
        scratch_shapes=[
            pltpu.VMEM((1, E), jnp.int32),
            pltpu.VMEM((1, E), jnp.float32),
            pltpu.VMEM((T, 2), jnp.int32),
            pltpu.VMEM((T, 2), jnp.int32),
        ],
        compiler_params=pltpu.CompilerParams(
            dimension_semantics=("arbitrary",)),
    )(h, W_router)


def _shared_body(E, NJ, hb_ref, wsg_ref, wsd_ref, e2_ref, w2_ref, wpgt_ref,
                 y_ref, acc):
    j = pl.program_id(1)

    @pl.when(j == 0)
    def _():
        acc[...] = jnp.zeros_like(acc)

    a = lax.dot_general(hb_ref[...], wsg_ref[...], (((1,), (0,)), ((), ())),
                        preferred_element_type=jnp.float32)
    a = jnp.maximum(a, 0.0).astype(jnp.bfloat16)
    acc[...] = acc[...] + lax.dot_general(
        a, wsd_ref[...], (((1,), (0,)), ((), ())),
        preferred_element_type=jnp.float32)

    @pl.when(j == NJ - 1)
    def _():
        hs = acc[...]
        p = lax.dot_general(hs.astype(jnp.bfloat16), wpgt_ref[...],
                            (((1,), (0,)), ((), ())),
                            preferred_element_type=jnp.float32)
        p = jax.nn.sigmoid(p)
        e2 = e2_ref[...]
        iota_e2 = lax.broadcasted_iota(jnp.int32, p.shape, 1)
        for k in range(2):
            psel = jnp.sum(
                jnp.where(e2[:, k:k + 1] == iota_e2, p, 0.0),
                axis=1, keepdims=True)
            psel = psel * w2_ref[:, k:k + 1]
            y_ref[:, k, :] = _pack_rows((psel * hs).astype(jnp.bfloat16))


def _shared(hb, Wsg, Wsd, e2, w2, Wpgt, TB, DSB):
    T, D = hb.shape
    DS = Wsg.shape[1]
    E = Wpgt.shape[1]
    NJ = DS // DSB
    grid = (T // TB, NJ)
    return pl.pallas_call(
        functools.partial(_shared_body, E, NJ),
        grid=grid,
        in_specs=[
            pl.BlockSpec((TB, D), lambda i, j: (i, 0)),
            pl.BlockSpec((D, DSB), lambda i, j: (0, j)),
            pl.BlockSpec((DSB, D), lambda i, j: (j, 0)),
            pl.BlockSpec((TB, 2), lambda i, j: (i, 0)),
            pl.BlockSpec((TB, 2), lambda i, j: (i, 0)),
            pl.BlockSpec((D, E), lambda i, j: (0, 0)),
        ],
        out_specs=pl.BlockSpec((TB, 2, D // 2), lambda i, j: (i, 0, 0)),
        out_shape=jax.ShapeDtypeStruct((T, 2, D // 2), jnp.int32),
        scratch_shapes=[pltpu.VMEM((TB, D), jnp.float32)],
        compiler_params=pltpu.CompilerParams(
            dimension_semantics=("parallel", "arbitrary")),
    )(hb, Wsg, Wsd, e2, w2, Wpgt)


_SC_CH = 32


def _sc_dispatch(src, idx3, out_rows, gather):
    D2 = src.shape[1]
    SL = D2 // 128
    NW, NCHUNK, CH = idx3.shape
    src3 = src.reshape(src.shape[0], SL, 128)
    mesh = plsc.VectorSubcoreMesh(core_axis_name="c", subcore_axis_name="s")

    @functools.partial(
        pl.kernel, mesh=mesh,
        out_type=jax.ShapeDtypeStruct((out_rows, SL, 128), jnp.int32),
        scratch_types=[
            pltpu.VMEM((NCHUNK, CH), jnp.int32),
            pltpu.VMEM((CH, SL, 128), jnp.int32),
            pltpu.VMEM((CH, SL, 128), jnp.int32),
            pltpu.SemaphoreType.DMA,
            pltpu.SemaphoreType.DMA,
            pltpu.SemaphoreType.DMA,
            pltpu.SemaphoreType.DMA,
        ],
    )
    def k(src_hbm, idx_hbm, out_hbm, idx_v, rows0, rows1,
          rs0, rs1, ws0, ws1):
        wid = lax.axis_index("s") * 2 + lax.axis_index("c")
        pltpu.sync_copy(idx_hbm.at[wid], idx_v)
        base = wid * (NCHUNK * CH)
        rows = (rows0, rows1)
        rsem = (rs0, rs1)
        wsem = (ws0, ws1)

        def rd(j):
            b = j % 2
            if gather:
                return pltpu.make_async_copy(
                    src_hbm.at[idx_v.at[j]], rows[b], rsem[b])
            return pltpu.make_async_copy(
                src_hbm.at[pl.ds(base + j * CH, CH)], rows[b], rsem[b])

        def wr(j):
            b = j % 2
            if gather:
                return pltpu.make_async_copy(
                    rows[b], out_hbm.at[pl.ds(base + j * CH, CH)], wsem[b])
            return pltpu.make_async_copy(
                rows[b], out_hbm.at[idx_v.at[j]], wsem[b])

        rd(0).start()
        for j in range(NCHUNK):
            if j + 1 < NCHUNK:
                if j >= 1:
                    wr(j - 1).wait()
                rd(j + 1).start()
            rd(j).wait()
            wr(j).start()
        wr(NCHUNK - 2).wait()
        wr(NCHUNK - 1).wait()

    out = k(src3, idx3)
    return out.reshape(out_rows, D2)


def _expert_body(be_ref, x_ref, wg_ref, wd_ref, o_ref):
    x = _unpack_rows(x_ref[...])
    wg = wg_ref[0].astype(jnp.bfloat16)
    t = lax.dot_general(x, wg, (((1,), (0,)), ((), ())),
                        preferred_element_type=jnp.float32)
    t = jnp.maximum(t, 0.0).astype(jnp.bfloat16)
    o = lax.dot_general(t, wd_ref[0].astype(jnp.bfloat16),
                        (((1,), (0,)), ((), ())),
                        preferred_element_type=jnp.float32)
    o_ref[...] = _pack_rows(o.astype(jnp.bfloat16))


def _expert_ffn(blk_e, x_d, W_gate, W_down, BT):
    NMAX, D2 = x_d.shape
    E, D, DP = W_gate.shape
    NB = NMAX // BT
    grid_spec = pltpu.PrefetchScalarGridSpec(
        num_scalar_prefetch=1,
        grid=(NB,),
        in_specs=[
            pl.BlockSpec((BT, D2), lambda i, be: (i, 0)),
            pl.BlockSpec((1, D, DP), lambda i, be: (be[i], 0, 0)),
            pl.BlockSpec((1, DP, D), lambda i, be: (be[i], 0, 0)),
        ],
        out_specs=pl.BlockSpec((BT, D2), lambda i, be: (i, 0)),
    )
    return pl.pallas_call(
        _expert_body,
        grid_spec=grid_spec,
        out_shape=jax.ShapeDtypeStruct((NMAX, D2), jnp.int32),
        compiler_params=pltpu.CompilerParams(
            dimension_semantics=("arbitrary",)),
    )(blk_e, x_d, W_gate, W_down)


def _combine_body(c_ref, o_ref):
    D2 = c_ref.shape[2]
    u0 = c_ref[:, 0, :]
    u1 = c_ref[:, 1, :]
    himask = jnp.int32(-65536)

    def lo_f(u):
        return lax.bitcast_convert_type(jnp.left_shift(u, 16), jnp.float32)

    def hi_f(u):
        return lax.bitcast_convert_type(u & himask, jnp.float32)

    o_ref[:, :D2] = lo_f(u0) + lo_f(u1)
    o_ref[:, D2:] = hi_f(u0) + hi_f(u1)


def _combine(comb, TB):
    T, K, D2 = comb.shape
    return pl.pallas_call(
        _combine_body,
        grid=(T // TB,),
        in_specs=[
            pl.BlockSpec((TB, K, D2), lambda i: (i, 0, 0)),
        ],
        out_specs=pl.BlockSpec((TB, D2 * 2), lambda i: (i, 0)),
        out_shape=jax.ShapeDtypeStruct((T, D2 * 2), jnp.float32),
        compiler_params=pltpu.CompilerParams(
            dimension_semantics=("parallel",)),
    )(comb)


def kernel(hidden_states, W_router, W_shared_gate, W_shared_down, W_gate,
           W_down, W_pgate):
    B, S, D = hidden_states.shape
    E = W_router.shape[1]
    DS = W_shared_gate.shape[1]
    T = B * S
    K = 2
    TK = T * K

    BT = 256 if TK >= 2048 else 8
    RB = 512 if T >= 2048 else T
    TB = 512 if T >= 2048 else T
    DSB = 512 if DS >= 2048 else DS
    NB = TK // BT + E
    NMAX = NB * BT

    h = hidden_states.reshape(T, D)
    hb = h.astype(jnp.bfloat16)
    logits, e2, w2, pflat, blk_e, aux = _router(
        hb, W_router.astype(jnp.bfloat16), RB, BT, NB)

    y = _shared(hb, W_shared_gate.astype(jnp.bfloat16),
                W_shared_down.astype(jnp.bfloat16), e2, w2,
                W_pgate.T.astype(jnp.bfloat16), TB, DSB)

    NW, CH = 32, _SC_CH
    D2 = D // 2
    idx3 = pflat.reshape(NW, TK // (NW * CH), CH)
    x_d = _sc_dispatch(y.reshape(TK, D2), idx3, NMAX, gather=False)
    out_d = _expert_ffn(blk_e.reshape(NB), x_d, W_gate, W_down, BT)
    comb = _sc_dispatch(out_d, idx3, TK, gather=True)
    final = _combine(comb.reshape(T, K, D2), TB)

    return final.reshape(B, S, D), logits, aux[0, 0]
